# baseline (speedup 1.0000x reference)
import functools

import jax
import jax.numpy as jnp
from jax import lax
from jax.experimental import pallas as pl
from jax.experimental.pallas import tpu as pltpu
from jax.experimental.pallas import tpu_sc as plsc

N = 10000
E = 320000
NC = 2
NS = 16
CH = 125
NB = 4
EW = E // (NC * NS)
NCH = EW // CH
RPS = N // NS
RSL = 632


def _sc_agg(y, src2d, dst2d, zf, zc, ones_c, with_count):
    n, f = y.shape
    mesh = plsc.VectorSubcoreMesh(core_axis_name="c", subcore_axis_name="s")
    out_type = [jax.ShapeDtypeStruct((n, f), jnp.float32),
                jax.ShapeDtypeStruct((n, f), jnp.float32)]
    scratch = (
        [pltpu.VMEM((NCH, CH), jnp.int32),
         pltpu.VMEM((NCH, CH), jnp.int32)]
        + [pltpu.VMEM((CH, f), jnp.float32)] * NB
        + [pltpu.VMEM_SHARED((n, f), jnp.float32)]
        + [pltpu.SemaphoreType.DMA] * (2 * NB)
    )
    if with_count:
        out_type += [jax.ShapeDtypeStruct((n, 16), jnp.float32),
                     jax.ShapeDtypeStruct((n, 16), jnp.float32)]
        scratch += [
            pltpu.VMEM((CH, 16), jnp.float32),
            pltpu.VMEM_SHARED((n, 16), jnp.float32),
        ]

    @functools.partial(
        pl.kernel, mesh=mesh, out_type=out_type, scratch_types=scratch,
        name=f"sc_seg_sum_f{f}_cnt{int(with_count)}",
        compiler_params=pltpu.CompilerParams(use_tc_tiling_on_sc=False),
    )
    def k(y_hbm, src_hbm, dst_hbm, zf_hbm, zc_hbm, ones_hbm, *rest):
        if with_count:
            (p0_hbm, p1_hbm, c0_hbm, c1_hbm, src_v, dst_v, *rr) = rest
        else:
            (p0_hbm, p1_hbm, src_v, dst_v, *rr) = rest
        rows = rr[:NB]
        acc_sh = rr[NB]
        gsem = rr[NB + 1:2 * NB + 1]
        ssem = rr[2 * NB + 1:3 * NB + 1]
        if with_count:
            ones_v, cnt_sh = rr[3 * NB + 1:]
        c = lax.axis_index("c")
        s = lax.axis_index("s")
        wid = c * NS + s

        row0 = pl.multiple_of(s * RPS - lax.rem(s, 8), 8)
        rsl = pl.ds(row0, RSL)

        pltpu.sync_copy(src_hbm.at[wid], src_v)
        pltpu.sync_copy(dst_hbm.at[wid], dst_v)
        pltpu.sync_copy(zf_hbm.at[rsl], acc_sh.at[rsl])
        if with_count:
            @pl.when(s == 0)
            def _():
                pltpu.sync_copy(zc_hbm, cnt_sh)
                pass
            pltpu.sync_copy(ones_hbm, ones_v)
        plsc.subcore_barrier()

        def gather_start(ch, b):
            pltpu.async_copy(y_hbm.at[src_v.at[ch]], rows[b], gsem[b])

        def gather_wait(ch, b):
            pltpu.make_async_copy(y_hbm.at[src_v.at[ch]], rows[b], gsem[b]).wait()

        def scatter_start(ch, b):
            pltpu.async_copy(rows[b], acc_sh.at[dst_v.at[ch]], ssem[b], add=True)
            if with_count:
                pltpu.sync_copy(ones_v, cnt_sh.at[dst_v.at[ch]], add=True)

        def scatter_wait(ch, b):
            pltpu.make_async_copy(rows[b], acc_sh.at[dst_v.at[ch]], ssem[b]).wait()

        for b in range(NB):
            gather_start(b, b)

        def body(j, _):
            j4 = NB * j
            for b in range(NB):
                gather_wait(j4 + b, b)
                scatter_start(j4 + b, b)
            for b in range(NB):
                scatter_wait(j4 + b, b)
                gather_start(j4 + NB + b, b)
            return 0

        lax.fori_loop(0, NCH // NB - 1, body, 0)
        last = NCH - NB
        for b in range(NB):
            gather_wait(last + b, b)
            scatter_start(last + b, b)
        for b in range(NB):
            scatter_wait(last + b, b)
        plsc.subcore_barrier()

        @pl.when(c == 0)
        def _():
            pltpu.sync_copy(acc_sh.at[rsl], p0_hbm.at[rsl])
            if with_count:
                @pl.when(s == 0)
                def _():
                    pltpu.sync_copy(cnt_sh, c0_hbm)
                    pass

        @pl.when(c == 1)
        def _():
            pltpu.sync_copy(acc_sh.at[rsl], p1_hbm.at[rsl])
            if with_count:
                @pl.when(s == 0)
                def _():
                    pltpu.sync_copy(cnt_sh, c1_hbm)
                    pass

    return k(y, src2d, dst2d, zf, zc, ones_c)


def _proj_body(x_ref, wl_ref, wr_ref, y_ref, r_ref):
    xb = x_ref[...]
    y_ref[...] = jnp.dot(xb, wl_ref[...], preferred_element_type=jnp.float32)
    r_ref[...] = jnp.dot(xb, wr_ref[...], preferred_element_type=jnp.float32)


def _proj(x, wl, wr):
    n, d = x.shape
    h = wl.shape[1]
    blk = 1000
    return pl.pallas_call(
        _proj_body,
        grid=(n // blk,),
        in_specs=[pl.BlockSpec((blk, d), lambda i: (i, 0)),
                  pl.BlockSpec((d, h), lambda i: (0, 0)),
                  pl.BlockSpec((d, h), lambda i: (0, 0))],
        out_specs=[pl.BlockSpec((blk, h), lambda i: (i, 0)),
                   pl.BlockSpec((blk, h), lambda i: (i, 0))],
        out_shape=[jax.ShapeDtypeStruct((n, h), jnp.float32),
                   jax.ShapeDtypeStruct((n, h), jnp.float32)],
    )(x, wl, wr)


def _norm_body(p0, p1, c0, c1, r, b, g, be, *rest):
    if len(rest) == 3:
        w2, x1_out, yz_out = rest
    else:
        w2 = None
        (x1_out,) = rest
    nn = p0.shape[0]
    cnt = jnp.maximum(c0[...] + c1[...], 1.0)
    h = (p0[...] + p1[...]) / cnt + b[...] + r[...]
    m = jnp.sum(h, axis=0, keepdims=True) * (1.0 / nn)
    d = h - m
    v = jnp.sum(d * d, axis=0, keepdims=True) * (1.0 / nn)
    hn = d * lax.rsqrt(v + 1e-5) * g[...] + be[...]
    x1 = jnp.maximum(hn, 0.0)
    x1_out[...] = x1
    if w2 is not None:
        yz_out[...] = jnp.dot(x1, w2[...], preferred_element_type=jnp.float32)


def _norm(p0, p1, c0, c1, r, b, g, be, w2=None):
    n, f = p0.shape
    args = [p0, p1, c0, c1, r, b.reshape(1, f), g.reshape(1, f), be.reshape(1, f)]
    out_shape = [jax.ShapeDtypeStruct((n, f), jnp.float32)]
    if w2 is not None:
        args.append(w2)
        out_shape.append(jax.ShapeDtypeStruct((n, w2.shape[1]), jnp.float32))
    outs = pl.pallas_call(
        functools.partial(_norm_body),
        in_specs=[pl.BlockSpec(a.shape, lambda: tuple(0 for _ in a.shape)) for a in args],
        out_specs=[pl.BlockSpec(o.shape, lambda: (0, 0)) for o in out_shape],
        out_shape=out_shape,
    )(*args)
    return outs


def _tail_body(q0, q1, c0, c1, r2, b2, g2, be2, w3, b3, w4, b4, w5, b5,
               x2_out, probs_out, logits_out, x2_s):
    i = pl.program_id(0)
    nn, blk = x2_s.shape[0], probs_out.shape[0]

    @pl.when(i == 0)
    def _():
        cnt = jnp.maximum(c0[...] + c1[...], 1.0)
        h = (q0[...] + q1[...]) / cnt + b2[...] + r2[...]
        m = jnp.sum(h, axis=0, keepdims=True) * (1.0 / nn)
        d = h - m
        v = jnp.sum(d * d, axis=0, keepdims=True) * (1.0 / nn)
        x2 = jnp.maximum(d * lax.rsqrt(v + 1e-5) * g2[...] + be2[...], 0.0)
        x2_out[...] = x2
        x2_s[...] = x2

    @pl.when(i > 0)
    def _():
        xb = x2_s[pl.ds((i - 1) * blk, blk), :]
        h = jnp.maximum(jnp.dot(xb, w3[...], preferred_element_type=jnp.float32) + b3[...], 0.0)
        h = jnp.maximum(jnp.dot(h.astype(jnp.bfloat16), w4[...], preferred_element_type=jnp.float32) + b4[...], 0.0)
        lg = jnp.dot(h, w5[...], preferred_element_type=jnp.float32) + b5[...]
        logits_out[...] = lg
        mx = jnp.max(lg, axis=1, keepdims=True)
        e = jnp.exp(lg - mx)
        probs_out[...] = e / jnp.sum(e, axis=1, keepdims=True)


def _tail(q0, q1, c0, c1, r2, b2, g2, be2, w3, b3, w4, b4, w5, b5):
    n, f = q0.shape
    mlp = w3.shape[1]
    c = w5.shape[1]
    blk = 1000
    def fixed(shape):
        return pl.BlockSpec(shape, lambda i: tuple(0 for _ in shape))
    mblk = lambda i: (jnp.maximum(i - 1, 0), 0)
    return pl.pallas_call(
        _tail_body,
        grid=(n // blk + 1,),
        in_specs=[fixed((n, f)), fixed((n, f)), fixed((n, 1)), fixed((n, 1)),
                  fixed((n, f)), fixed((1, f)), fixed((1, f)), fixed((1, f)),
                  fixed((f, mlp)), fixed((1, mlp)), fixed((mlp, mlp)),
                  fixed((1, mlp)), fixed((mlp, c)), fixed((1, c))],
        out_specs=[fixed((n, f)),
                   pl.BlockSpec((blk, c), mblk),
                   pl.BlockSpec((blk, c), mblk)],
        out_shape=[jax.ShapeDtypeStruct((n, f), jnp.float32),
                   jax.ShapeDtypeStruct((n, c), jnp.float32),
                   jax.ShapeDtypeStruct((n, c), jnp.float32)],
        scratch_shapes=[pltpu.VMEM((n, f), jnp.float32)],
    )(q0, q1, c0, c1, r2, b2.reshape(1, f), g2.reshape(1, f), be2.reshape(1, f),
      w3, b3.reshape(1, mlp), w4.astype(jnp.bfloat16), b4.reshape(1, mlp),
      w5, b5.reshape(1, c))


def _mlp_body(x2, w3, b3, w4, b4, w5, b5, probs_out, logits_out):
    h = jnp.maximum(jnp.dot(x2[...], w3[...], preferred_element_type=jnp.float32) + b3[...], 0.0)
    h = jnp.maximum(jnp.dot(h.astype(jnp.bfloat16), w4[...], preferred_element_type=jnp.float32) + b4[...], 0.0)
    lg = jnp.dot(h, w5[...], preferred_element_type=jnp.float32) + b5[...]
    logits_out[...] = lg
    mx = jnp.max(lg, axis=1, keepdims=True)
    e = jnp.exp(lg - mx)
    probs_out[...] = e / jnp.sum(e, axis=1, keepdims=True)


def _mlp(x2, w3, b3, w4, b4, w5, b5):
    n, f = x2.shape
    mlp = w3.shape[1]
    c = w5.shape[1]
    blk = 1000
    return pl.pallas_call(
        _mlp_body,
        grid=(n // blk,),
        in_specs=[pl.BlockSpec((blk, f), lambda i: (i, 0)),
                  pl.BlockSpec((f, mlp), lambda i: (0, 0)),
                  pl.BlockSpec((1, mlp), lambda i: (0, 0)),
                  pl.BlockSpec((mlp, mlp), lambda i: (0, 0)),
                  pl.BlockSpec((1, mlp), lambda i: (0, 0)),
                  pl.BlockSpec((mlp, c), lambda i: (0, 0)),
                  pl.BlockSpec((1, c), lambda i: (0, 0))],
        out_specs=[pl.BlockSpec((blk, c), lambda i: (i, 0)),
                   pl.BlockSpec((blk, c), lambda i: (i, 0))],
        out_shape=[jax.ShapeDtypeStruct((n, c), jnp.float32),
                   jax.ShapeDtypeStruct((n, c), jnp.float32)],
    )(x2, w3, b3.reshape(1, mlp), w4.astype(jnp.bfloat16), b4.reshape(1, mlp),
      w5, b5.reshape(1, c))


def kernel(x, edge_index, W1_l, b1_l, W1_r, gamma1, beta1, W2_l, b2_l, W2_r,
           gamma2, beta2, W3, b3, W4, b4, W5, b5):
    src2d = edge_index[0].reshape(NC * NS, NCH, CH)
    dst2d = edge_index[1].reshape(NC * NS, NCH, CH)
    h1 = W1_l.shape[1]
    h2 = W2_l.shape[1]
    zc = jnp.zeros((N, 16), jnp.float32)
    ones_c = jnp.ones((CH, 16), jnp.float32)

    y1, r1 = _proj(x, W1_l, W1_r)
    p0, p1, c0, c1 = _sc_agg(y1, src2d, dst2d, jnp.zeros((N, h1), jnp.float32),
                             zc, ones_c, with_count=True)
    w2cat = jnp.concatenate([W2_l, W2_r], axis=1)
    c0 = c0[:, :1]
    c1 = c1[:, :1]
    x1, yz = _norm(p0, p1, c0, c1, r1, b1_l, gamma1, beta1, w2cat)
    y2 = yz[:, :h2]
    r2 = yz[:, h2:]
    q0, q1 = _sc_agg(y2, src2d, dst2d, jnp.zeros((N, h2), jnp.float32),
                     zc, ones_c, with_count=False)
    x2, probs, logits = _tail(q0, q1, c0, c1, r2, b2_l, gamma2, beta2,
                              W3, b3, W4, b4, W5, b5)
    return probs, logits, x1, x2


# speedup vs baseline: 1.0606x; 1.0606x over previous
import functools

import jax
import jax.numpy as jnp
from jax import lax
from jax.experimental import pallas as pl
from jax.experimental.pallas import tpu as pltpu
from jax.experimental.pallas import tpu_sc as plsc

N = 10000
E = 320000
NC = 2
NS = 16
CH = 125
NB = 4
EW = E // (NC * NS)
NCH = EW // CH
RPS = N // NS
RSL = 632


def _sc_agg(y, src2d, dst2d, zf, zc, ones_c, with_count):
    n, f = y.shape
    mesh = plsc.VectorSubcoreMesh(core_axis_name="c", subcore_axis_name="s")
    out_type = [jax.ShapeDtypeStruct((n, f), jnp.float32),
                jax.ShapeDtypeStruct((n, f), jnp.float32)]
    scratch = (
        [pltpu.VMEM((NCH, CH), jnp.int32),
         pltpu.VMEM((NCH, CH), jnp.int32)]
        + [pltpu.VMEM((CH, f), jnp.float32)] * NB
        + [pltpu.VMEM_SHARED((n, f), jnp.float32)]
        + [pltpu.SemaphoreType.DMA] * (2 * NB)
    )
    if with_count:
        out_type += [jax.ShapeDtypeStruct((n, 16), jnp.float32),
                     jax.ShapeDtypeStruct((n, 16), jnp.float32)]
        scratch += [
            pltpu.VMEM((CH, 16), jnp.float32),
            pltpu.VMEM_SHARED((n, 16), jnp.float32),
            pltpu.SemaphoreType.DMA,
        ]

    @functools.partial(
        pl.kernel, mesh=mesh, out_type=out_type, scratch_types=scratch,
        name=f"sc_seg_sum_f{f}_cnt{int(with_count)}",
        compiler_params=pltpu.CompilerParams(use_tc_tiling_on_sc=False),
    )
    def k(y_hbm, src_hbm, dst_hbm, zf_hbm, zc_hbm, ones_hbm, *rest):
        if with_count:
            (p0_hbm, p1_hbm, c0_hbm, c1_hbm, src_v, dst_v, *rr) = rest
        else:
            (p0_hbm, p1_hbm, src_v, dst_v, *rr) = rest
        rows = rr[:NB]
        acc_sh = rr[NB]
        gsem = rr[NB + 1:2 * NB + 1]
        ssem = rr[2 * NB + 1:3 * NB + 1]
        if with_count:
            ones_v, cnt_sh, csem = rr[3 * NB + 1:]
        c = lax.axis_index("c")
        s = lax.axis_index("s")
        wid = c * NS + s

        row0 = pl.multiple_of(s * RPS - lax.rem(s, 8), 8)
        rsl = pl.ds(row0, RSL)

        pltpu.sync_copy(src_hbm.at[wid], src_v)
        pltpu.sync_copy(dst_hbm.at[wid], dst_v)
        pltpu.sync_copy(zf_hbm.at[rsl], acc_sh.at[rsl])
        if with_count:
            @pl.when(s == 0)
            def _():
                pltpu.sync_copy(zc_hbm, cnt_sh)
                pass
            pltpu.sync_copy(ones_hbm, ones_v)
        plsc.subcore_barrier()

        def gather_start(ch, b):
            pltpu.async_copy(y_hbm.at[src_v.at[ch]], rows[b], gsem[b])

        def gather_wait(ch, b):
            pltpu.make_async_copy(y_hbm.at[src_v.at[ch]], rows[b], gsem[b]).wait()

        def scatter_start(ch, b):
            pltpu.async_copy(rows[b], acc_sh.at[dst_v.at[ch]], ssem[b], add=True)
            if with_count:
                pltpu.async_copy(ones_v, cnt_sh.at[dst_v.at[ch]], csem, add=True)

        def scatter_wait(ch, b):
            pltpu.make_async_copy(rows[b], acc_sh.at[dst_v.at[ch]], ssem[b]).wait()
            if with_count:
                pltpu.make_async_copy(ones_v, cnt_sh.at[dst_v.at[ch]], csem).wait()

        for b in range(NB):
            gather_start(b, b)

        def body(j, _):
            j4 = NB * j
            for b in range(NB):
                gather_wait(j4 + b, b)
                scatter_start(j4 + b, b)
            for b in range(NB):
                scatter_wait(j4 + b, b)
                gather_start(j4 + NB + b, b)
            return 0

        lax.fori_loop(0, NCH // NB - 1, body, 0)
        last = NCH - NB
        for b in range(NB):
            gather_wait(last + b, b)
            scatter_start(last + b, b)
        for b in range(NB):
            scatter_wait(last + b, b)
        plsc.subcore_barrier()

        @pl.when(c == 0)
        def _():
            pltpu.sync_copy(acc_sh.at[rsl], p0_hbm.at[rsl])
            if with_count:
                @pl.when(s == 0)
                def _():
                    pltpu.sync_copy(cnt_sh, c0_hbm)
                    pass

        @pl.when(c == 1)
        def _():
            pltpu.sync_copy(acc_sh.at[rsl], p1_hbm.at[rsl])
            if with_count:
                @pl.when(s == 0)
                def _():
                    pltpu.sync_copy(cnt_sh, c1_hbm)
                    pass

    return k(y, src2d, dst2d, zf, zc, ones_c)


def _proj_body(x_ref, wl_ref, wr_ref, y_ref, r_ref):
    xb = x_ref[...]
    y_ref[...] = jnp.dot(xb, wl_ref[...], preferred_element_type=jnp.float32)
    r_ref[...] = jnp.dot(xb, wr_ref[...], preferred_element_type=jnp.float32)


def _proj(x, wl, wr):
    n, d = x.shape
    h = wl.shape[1]
    blk = 1000
    return pl.pallas_call(
        _proj_body,
        grid=(n // blk,),
        in_specs=[pl.BlockSpec((blk, d), lambda i: (i, 0)),
                  pl.BlockSpec((d, h), lambda i: (0, 0)),
                  pl.BlockSpec((d, h), lambda i: (0, 0))],
        out_specs=[pl.BlockSpec((blk, h), lambda i: (i, 0)),
                   pl.BlockSpec((blk, h), lambda i: (i, 0))],
        out_shape=[jax.ShapeDtypeStruct((n, h), jnp.float32),
                   jax.ShapeDtypeStruct((n, h), jnp.float32)],
    )(x, wl, wr)


def _norm_body(p0, p1, c0, c1, r, b, g, be, *rest):
    if len(rest) == 3:
        w2, x1_out, yz_out = rest
    else:
        w2 = None
        (x1_out,) = rest
    nn = p0.shape[0]
    cnt = jnp.maximum(c0[...] + c1[...], 1.0)
    h = (p0[...] + p1[...]) / cnt + b[...] + r[...]
    m = jnp.sum(h, axis=0, keepdims=True) * (1.0 / nn)
    d = h - m
    v = jnp.sum(d * d, axis=0, keepdims=True) * (1.0 / nn)
    hn = d * lax.rsqrt(v + 1e-5) * g[...] + be[...]
    x1 = jnp.maximum(hn, 0.0)
    x1_out[...] = x1
    if w2 is not None:
        yz_out[...] = jnp.dot(x1, w2[...], preferred_element_type=jnp.float32)


def _norm(p0, p1, c0, c1, r, b, g, be, w2=None):
    n, f = p0.shape
    args = [p0, p1, c0, c1, r, b.reshape(1, f), g.reshape(1, f), be.reshape(1, f)]
    out_shape = [jax.ShapeDtypeStruct((n, f), jnp.float32)]
    if w2 is not None:
        args.append(w2)
        out_shape.append(jax.ShapeDtypeStruct((n, w2.shape[1]), jnp.float32))
    outs = pl.pallas_call(
        functools.partial(_norm_body),
        in_specs=[pl.BlockSpec(a.shape, lambda: tuple(0 for _ in a.shape)) for a in args],
        out_specs=[pl.BlockSpec(o.shape, lambda: (0, 0)) for o in out_shape],
        out_shape=out_shape,
    )(*args)
    return outs


def _tail_body(q0, q1, c0, c1, r2, b2, g2, be2, w3, b3, w4, b4, w5, b5,
               x2_out, probs_out, logits_out, x2_s):
    i = pl.program_id(0)
    nn, blk = x2_s.shape[0], probs_out.shape[0]

    @pl.when(i == 0)
    def _():
        cnt = jnp.maximum(c0[...] + c1[...], 1.0)
        h = (q0[...] + q1[...]) / cnt + b2[...] + r2[...]
        m = jnp.sum(h, axis=0, keepdims=True) * (1.0 / nn)
        d = h - m
        v = jnp.sum(d * d, axis=0, keepdims=True) * (1.0 / nn)
        x2 = jnp.maximum(d * lax.rsqrt(v + 1e-5) * g2[...] + be2[...], 0.0)
        x2_out[...] = x2
        x2_s[...] = x2

    @pl.when(i > 0)
    def _():
        xb = x2_s[pl.ds((i - 1) * blk, blk), :]
        h = jnp.maximum(jnp.dot(xb, w3[...], preferred_element_type=jnp.float32) + b3[...], 0.0)
        h = jnp.maximum(jnp.dot(h.astype(jnp.bfloat16), w4[...], preferred_element_type=jnp.float32) + b4[...], 0.0)
        lg = jnp.dot(h, w5[...], preferred_element_type=jnp.float32) + b5[...]
        logits_out[...] = lg
        mx = jnp.max(lg, axis=1, keepdims=True)
        e = jnp.exp(lg - mx)
        probs_out[...] = e / jnp.sum(e, axis=1, keepdims=True)


def _tail(q0, q1, c0, c1, r2, b2, g2, be2, w3, b3, w4, b4, w5, b5):
    n, f = q0.shape
    mlp = w3.shape[1]
    c = w5.shape[1]
    blk = 1000
    def fixed(shape):
        return pl.BlockSpec(shape, lambda i: tuple(0 for _ in shape))
    mblk = lambda i: (jnp.maximum(i - 1, 0), 0)
    return pl.pallas_call(
        _tail_body,
        grid=(n // blk + 1,),
        in_specs=[fixed((n, f)), fixed((n, f)), fixed((n, 1)), fixed((n, 1)),
                  fixed((n, f)), fixed((1, f)), fixed((1, f)), fixed((1, f)),
                  fixed((f, mlp)), fixed((1, mlp)), fixed((mlp, mlp)),
                  fixed((1, mlp)), fixed((mlp, c)), fixed((1, c))],
        out_specs=[fixed((n, f)),
                   pl.BlockSpec((blk, c), mblk),
                   pl.BlockSpec((blk, c), mblk)],
        out_shape=[jax.ShapeDtypeStruct((n, f), jnp.float32),
                   jax.ShapeDtypeStruct((n, c), jnp.float32),
                   jax.ShapeDtypeStruct((n, c), jnp.float32)],
        scratch_shapes=[pltpu.VMEM((n, f), jnp.float32)],
    )(q0, q1, c0, c1, r2, b2.reshape(1, f), g2.reshape(1, f), be2.reshape(1, f),
      w3, b3.reshape(1, mlp), w4.astype(jnp.bfloat16), b4.reshape(1, mlp),
      w5, b5.reshape(1, c))


def _mlp_body(x2, w3, b3, w4, b4, w5, b5, probs_out, logits_out):
    h = jnp.maximum(jnp.dot(x2[...], w3[...], preferred_element_type=jnp.float32) + b3[...], 0.0)
    h = jnp.maximum(jnp.dot(h.astype(jnp.bfloat16), w4[...], preferred_element_type=jnp.float32) + b4[...], 0.0)
    lg = jnp.dot(h, w5[...], preferred_element_type=jnp.float32) + b5[...]
    logits_out[...] = lg
    mx = jnp.max(lg, axis=1, keepdims=True)
    e = jnp.exp(lg - mx)
    probs_out[...] = e / jnp.sum(e, axis=1, keepdims=True)


def _mlp(x2, w3, b3, w4, b4, w5, b5):
    n, f = x2.shape
    mlp = w3.shape[1]
    c = w5.shape[1]
    blk = 1000
    return pl.pallas_call(
        _mlp_body,
        grid=(n // blk,),
        in_specs=[pl.BlockSpec((blk, f), lambda i: (i, 0)),
                  pl.BlockSpec((f, mlp), lambda i: (0, 0)),
                  pl.BlockSpec((1, mlp), lambda i: (0, 0)),
                  pl.BlockSpec((mlp, mlp), lambda i: (0, 0)),
                  pl.BlockSpec((1, mlp), lambda i: (0, 0)),
                  pl.BlockSpec((mlp, c), lambda i: (0, 0)),
                  pl.BlockSpec((1, c), lambda i: (0, 0))],
        out_specs=[pl.BlockSpec((blk, c), lambda i: (i, 0)),
                   pl.BlockSpec((blk, c), lambda i: (i, 0))],
        out_shape=[jax.ShapeDtypeStruct((n, c), jnp.float32),
                   jax.ShapeDtypeStruct((n, c), jnp.float32)],
    )(x2, w3, b3.reshape(1, mlp), w4.astype(jnp.bfloat16), b4.reshape(1, mlp),
      w5, b5.reshape(1, c))


def kernel(x, edge_index, W1_l, b1_l, W1_r, gamma1, beta1, W2_l, b2_l, W2_r,
           gamma2, beta2, W3, b3, W4, b4, W5, b5):
    src2d = edge_index[0].reshape(NC * NS, NCH, CH)
    dst2d = edge_index[1].reshape(NC * NS, NCH, CH)
    h1 = W1_l.shape[1]
    h2 = W2_l.shape[1]
    zc = jnp.zeros((N, 16), jnp.float32)
    ones_c = jnp.ones((CH, 16), jnp.float32)

    y1, r1 = _proj(x, W1_l, W1_r)
    p0, p1, c0, c1 = _sc_agg(y1, src2d, dst2d, jnp.zeros((N, h1), jnp.float32),
                             zc, ones_c, with_count=True)
    w2cat = jnp.concatenate([W2_l, W2_r], axis=1)
    c0 = c0[:, :1]
    c1 = c1[:, :1]
    x1, yz = _norm(p0, p1, c0, c1, r1, b1_l, gamma1, beta1, w2cat)
    y2 = yz[:, :h2]
    r2 = yz[:, h2:]
    q0, q1 = _sc_agg(y2, src2d, dst2d, jnp.zeros((N, h2), jnp.float32),
                     zc, ones_c, with_count=False)
    x2, probs, logits = _tail(q0, q1, c0, c1, r2, b2_l, gamma2, beta2,
                              W3, b3, W4, b4, W5, b5)
    return probs, logits, x1, x2


# speedup vs baseline: 1.0750x; 1.0135x over previous
import functools

import jax
import jax.numpy as jnp
from jax import lax
from jax.experimental import pallas as pl
from jax.experimental.pallas import tpu as pltpu
from jax.experimental.pallas import tpu_sc as plsc

N = 10000
E = 320000
NC = 2
NS = 16
CH = 125
NB = 4
EW = E // (NC * NS)
NCH = EW // CH
RPS = N // NS
RSL = 632


def _sc_agg(y, src2d, dst2d, zf, zc, ones_c, with_count):
    n, f = y.shape
    NB = 4 if with_count else 8
    mesh = plsc.VectorSubcoreMesh(core_axis_name="c", subcore_axis_name="s")
    out_type = [jax.ShapeDtypeStruct((n, f), jnp.float32),
                jax.ShapeDtypeStruct((n, f), jnp.float32)]
    scratch = (
        [pltpu.VMEM((NCH, CH), jnp.int32),
         pltpu.VMEM((NCH, CH), jnp.int32)]
        + [pltpu.VMEM((CH, f), jnp.float32)] * NB
        + [pltpu.VMEM_SHARED((n, f), jnp.float32)]
        + [pltpu.SemaphoreType.DMA] * (2 * NB)
    )
    if with_count:
        out_type += [jax.ShapeDtypeStruct((n, 16), jnp.float32),
                     jax.ShapeDtypeStruct((n, 16), jnp.float32)]
        scratch += [
            pltpu.VMEM((CH, 16), jnp.float32),
            pltpu.VMEM_SHARED((n, 16), jnp.float32),
            pltpu.SemaphoreType.DMA,
        ]

    @functools.partial(
        pl.kernel, mesh=mesh, out_type=out_type, scratch_types=scratch,
        name=f"sc_seg_sum_f{f}_cnt{int(with_count)}",
        compiler_params=pltpu.CompilerParams(use_tc_tiling_on_sc=False),
    )
    def k(y_hbm, src_hbm, dst_hbm, zf_hbm, zc_hbm, ones_hbm, *rest):
        if with_count:
            (p0_hbm, p1_hbm, c0_hbm, c1_hbm, src_v, dst_v, *rr) = rest
        else:
            (p0_hbm, p1_hbm, src_v, dst_v, *rr) = rest
        rows = rr[:NB]
        acc_sh = rr[NB]
        gsem = rr[NB + 1:2 * NB + 1]
        ssem = rr[2 * NB + 1:3 * NB + 1]
        if with_count:
            ones_v, cnt_sh, csem = rr[3 * NB + 1:]
        c = lax.axis_index("c")
        s = lax.axis_index("s")
        wid = c * NS + s

        row0 = pl.multiple_of(s * RPS - lax.rem(s, 8), 8)
        rsl = pl.ds(row0, RSL)

        pltpu.sync_copy(src_hbm.at[wid], src_v)
        pltpu.sync_copy(dst_hbm.at[wid], dst_v)
        pltpu.sync_copy(zf_hbm.at[rsl], acc_sh.at[rsl])
        if with_count:
            @pl.when(s == 0)
            def _():
                pltpu.sync_copy(zc_hbm, cnt_sh)
                pass
            pltpu.sync_copy(ones_hbm, ones_v)
        plsc.subcore_barrier()

        def gather_start(ch, b):
            pltpu.async_copy(y_hbm.at[src_v.at[ch]], rows[b], gsem[b])

        def gather_wait(ch, b):
            pltpu.make_async_copy(y_hbm.at[src_v.at[ch]], rows[b], gsem[b]).wait()

        def scatter_start(ch, b):
            pltpu.async_copy(rows[b], acc_sh.at[dst_v.at[ch]], ssem[b], add=True)
            if with_count:
                pltpu.async_copy(ones_v, cnt_sh.at[dst_v.at[ch]], csem, add=True)

        def scatter_wait(ch, b):
            pltpu.make_async_copy(rows[b], acc_sh.at[dst_v.at[ch]], ssem[b]).wait()
            if with_count:
                pltpu.make_async_copy(ones_v, cnt_sh.at[dst_v.at[ch]], csem).wait()

        for b in range(NB):
            gather_start(b, b)

        def body(j, _):
            j4 = NB * j
            for b in range(NB):
                gather_wait(j4 + b, b)
                scatter_start(j4 + b, b)
            for b in range(NB):
                scatter_wait(j4 + b, b)
                gather_start(j4 + NB + b, b)
            return 0

        lax.fori_loop(0, NCH // NB - 1, body, 0)
        last = NCH - NB
        for b in range(NB):
            gather_wait(last + b, b)
            scatter_start(last + b, b)
        for b in range(NB):
            scatter_wait(last + b, b)
        plsc.subcore_barrier()

        @pl.when(c == 0)
        def _():
            pltpu.sync_copy(acc_sh.at[rsl], p0_hbm.at[rsl])
            if with_count:
                @pl.when(s == 0)
                def _():
                    pltpu.sync_copy(cnt_sh, c0_hbm)
                    pass

        @pl.when(c == 1)
        def _():
            pltpu.sync_copy(acc_sh.at[rsl], p1_hbm.at[rsl])
            if with_count:
                @pl.when(s == 0)
                def _():
                    pltpu.sync_copy(cnt_sh, c1_hbm)
                    pass

    return k(y, src2d, dst2d, zf, zc, ones_c)


def _proj_body(x_ref, wl_ref, wr_ref, y_ref, r_ref):
    xb = x_ref[...]
    y_ref[...] = jnp.dot(xb, wl_ref[...], preferred_element_type=jnp.float32)
    r_ref[...] = jnp.dot(xb, wr_ref[...], preferred_element_type=jnp.float32)


def _proj(x, wl, wr):
    n, d = x.shape
    h = wl.shape[1]
    blk = 1000
    return pl.pallas_call(
        _proj_body,
        grid=(n // blk,),
        in_specs=[pl.BlockSpec((blk, d), lambda i: (i, 0)),
                  pl.BlockSpec((d, h), lambda i: (0, 0)),
                  pl.BlockSpec((d, h), lambda i: (0, 0))],
        out_specs=[pl.BlockSpec((blk, h), lambda i: (i, 0)),
                   pl.BlockSpec((blk, h), lambda i: (i, 0))],
        out_shape=[jax.ShapeDtypeStruct((n, h), jnp.float32),
                   jax.ShapeDtypeStruct((n, h), jnp.float32)],
    )(x, wl, wr)


def _norm_body(p0, p1, c0, c1, r, b, g, be, *rest):
    if len(rest) == 3:
        w2, x1_out, yz_out = rest
    else:
        w2 = None
        (x1_out,) = rest
    nn = p0.shape[0]
    cnt = jnp.maximum(c0[...] + c1[...], 1.0)
    h = (p0[...] + p1[...]) / cnt + b[...] + r[...]
    m = jnp.sum(h, axis=0, keepdims=True) * (1.0 / nn)
    d = h - m
    v = jnp.sum(d * d, axis=0, keepdims=True) * (1.0 / nn)
    hn = d * lax.rsqrt(v + 1e-5) * g[...] + be[...]
    x1 = jnp.maximum(hn, 0.0)
    x1_out[...] = x1
    if w2 is not None:
        yz_out[...] = jnp.dot(x1, w2[...], preferred_element_type=jnp.float32)


def _norm(p0, p1, c0, c1, r, b, g, be, w2=None):
    n, f = p0.shape
    args = [p0, p1, c0, c1, r, b.reshape(1, f), g.reshape(1, f), be.reshape(1, f)]
    out_shape = [jax.ShapeDtypeStruct((n, f), jnp.float32)]
    if w2 is not None:
        args.append(w2)
        out_shape.append(jax.ShapeDtypeStruct((n, w2.shape[1]), jnp.float32))
    outs = pl.pallas_call(
        functools.partial(_norm_body),
        in_specs=[pl.BlockSpec(a.shape, lambda: tuple(0 for _ in a.shape)) for a in args],
        out_specs=[pl.BlockSpec(o.shape, lambda: (0, 0)) for o in out_shape],
        out_shape=out_shape,
    )(*args)
    return outs


def _tail_body(q0, q1, c0, c1, r2, b2, g2, be2, w3, b3, w4, b4, w5, b5,
               x2_out, probs_out, logits_out, x2_s):
    i = pl.program_id(0)
    nn, blk = x2_s.shape[0], probs_out.shape[0]

    @pl.when(i == 0)
    def _():
        cnt = jnp.maximum(c0[...] + c1[...], 1.0)
        h = (q0[...] + q1[...]) / cnt + b2[...] + r2[...]
        m = jnp.sum(h, axis=0, keepdims=True) * (1.0 / nn)
        d = h - m
        v = jnp.sum(d * d, axis=0, keepdims=True) * (1.0 / nn)
        x2 = jnp.maximum(d * lax.rsqrt(v + 1e-5) * g2[...] + be2[...], 0.0)
        x2_out[...] = x2
        x2_s[...] = x2

    @pl.when(i > 0)
    def _():
        xb = x2_s[pl.ds((i - 1) * blk, blk), :]
        h = jnp.maximum(jnp.dot(xb, w3[...], preferred_element_type=jnp.float32) + b3[...], 0.0)
        h = jnp.maximum(jnp.dot(h.astype(jnp.bfloat16), w4[...], preferred_element_type=jnp.float32) + b4[...], 0.0)
        lg = jnp.dot(h, w5[...], preferred_element_type=jnp.float32) + b5[...]
        logits_out[...] = lg
        mx = jnp.max(lg, axis=1, keepdims=True)
        e = jnp.exp(lg - mx)
        probs_out[...] = e / jnp.sum(e, axis=1, keepdims=True)


def _tail(q0, q1, c0, c1, r2, b2, g2, be2, w3, b3, w4, b4, w5, b5):
    n, f = q0.shape
    mlp = w3.shape[1]
    c = w5.shape[1]
    blk = 1000
    def fixed(shape):
        return pl.BlockSpec(shape, lambda i: tuple(0 for _ in shape))
    mblk = lambda i: (jnp.maximum(i - 1, 0), 0)
    return pl.pallas_call(
        _tail_body,
        grid=(n // blk + 1,),
        in_specs=[fixed((n, f)), fixed((n, f)), fixed((n, 1)), fixed((n, 1)),
                  fixed((n, f)), fixed((1, f)), fixed((1, f)), fixed((1, f)),
                  fixed((f, mlp)), fixed((1, mlp)), fixed((mlp, mlp)),
                  fixed((1, mlp)), fixed((mlp, c)), fixed((1, c))],
        out_specs=[fixed((n, f)),
                   pl.BlockSpec((blk, c), mblk),
                   pl.BlockSpec((blk, c), mblk)],
        out_shape=[jax.ShapeDtypeStruct((n, f), jnp.float32),
                   jax.ShapeDtypeStruct((n, c), jnp.float32),
                   jax.ShapeDtypeStruct((n, c), jnp.float32)],
        scratch_shapes=[pltpu.VMEM((n, f), jnp.float32)],
    )(q0, q1, c0, c1, r2, b2.reshape(1, f), g2.reshape(1, f), be2.reshape(1, f),
      w3, b3.reshape(1, mlp), w4.astype(jnp.bfloat16), b4.reshape(1, mlp),
      w5, b5.reshape(1, c))


def _mlp_body(x2, w3, b3, w4, b4, w5, b5, probs_out, logits_out):
    h = jnp.maximum(jnp.dot(x2[...], w3[...], preferred_element_type=jnp.float32) + b3[...], 0.0)
    h = jnp.maximum(jnp.dot(h.astype(jnp.bfloat16), w4[...], preferred_element_type=jnp.float32) + b4[...], 0.0)
    lg = jnp.dot(h, w5[...], preferred_element_type=jnp.float32) + b5[...]
    logits_out[...] = lg
    mx = jnp.max(lg, axis=1, keepdims=True)
    e = jnp.exp(lg - mx)
    probs_out[...] = e / jnp.sum(e, axis=1, keepdims=True)


def _mlp(x2, w3, b3, w4, b4, w5, b5):
    n, f = x2.shape
    mlp = w3.shape[1]
    c = w5.shape[1]
    blk = 1000
    return pl.pallas_call(
        _mlp_body,
        grid=(n // blk,),
        in_specs=[pl.BlockSpec((blk, f), lambda i: (i, 0)),
                  pl.BlockSpec((f, mlp), lambda i: (0, 0)),
                  pl.BlockSpec((1, mlp), lambda i: (0, 0)),
                  pl.BlockSpec((mlp, mlp), lambda i: (0, 0)),
                  pl.BlockSpec((1, mlp), lambda i: (0, 0)),
                  pl.BlockSpec((mlp, c), lambda i: (0, 0)),
                  pl.BlockSpec((1, c), lambda i: (0, 0))],
        out_specs=[pl.BlockSpec((blk, c), lambda i: (i, 0)),
                   pl.BlockSpec((blk, c), lambda i: (i, 0))],
        out_shape=[jax.ShapeDtypeStruct((n, c), jnp.float32),
                   jax.ShapeDtypeStruct((n, c), jnp.float32)],
    )(x2, w3, b3.reshape(1, mlp), w4.astype(jnp.bfloat16), b4.reshape(1, mlp),
      w5, b5.reshape(1, c))


def kernel(x, edge_index, W1_l, b1_l, W1_r, gamma1, beta1, W2_l, b2_l, W2_r,
           gamma2, beta2, W3, b3, W4, b4, W5, b5):
    src2d = edge_index[0].reshape(NC * NS, NCH, CH)
    dst2d = edge_index[1].reshape(NC * NS, NCH, CH)
    h1 = W1_l.shape[1]
    h2 = W2_l.shape[1]
    zc = jnp.zeros((N, 16), jnp.float32)
    ones_c = jnp.ones((CH, 16), jnp.float32)

    y1, r1 = _proj(x, W1_l, W1_r)
    p0, p1, c0, c1 = _sc_agg(y1, src2d, dst2d, jnp.zeros((N, h1), jnp.float32),
                             zc, ones_c, with_count=True)
    w2cat = jnp.concatenate([W2_l, W2_r], axis=1)
    c0 = c0[:, :1]
    c1 = c1[:, :1]
    x1, yz = _norm(p0, p1, c0, c1, r1, b1_l, gamma1, beta1, w2cat)
    y2 = yz[:, :h2]
    r2 = yz[:, h2:]
    q0, q1 = _sc_agg(y2, src2d, dst2d, jnp.zeros((N, h2), jnp.float32),
                     zc, ones_c, with_count=False)
    x2, probs, logits = _tail(q0, q1, c0, c1, r2, b2_l, gamma2, beta2,
                              W3, b3, W4, b4, W5, b5)
    return probs, logits, x1, x2


# speedup vs baseline: 1.0755x; 1.0005x over previous
import functools

import jax
import jax.numpy as jnp
from jax import lax
from jax.experimental import pallas as pl
from jax.experimental.pallas import tpu as pltpu
from jax.experimental.pallas import tpu_sc as plsc

N = 10000
E = 320000
NC = 2
NS = 16
CH = 125
NB = 4
EW = E // (NC * NS)
NCH = EW // CH
RPS = N // NS
RSL = 632


def _sc_agg(y, src2d, dst2d, zf, zc, ones_c, with_count):
    n, f = y.shape
    NB = 4 if with_count else 8
    mesh = plsc.VectorSubcoreMesh(core_axis_name="c", subcore_axis_name="s")
    out_type = [jax.ShapeDtypeStruct((n, f), jnp.float32),
                jax.ShapeDtypeStruct((n, f), jnp.float32)]
    scratch = (
        [pltpu.VMEM((NCH, CH), jnp.int32),
         pltpu.VMEM((NCH, CH), jnp.int32)]
        + [pltpu.VMEM((CH, f), jnp.float32)] * NB
        + [pltpu.VMEM_SHARED((n, f), jnp.float32)]
        + [pltpu.SemaphoreType.DMA] * (2 * NB)
    )
    if with_count:
        out_type += [jax.ShapeDtypeStruct((n, 16), jnp.float32),
                     jax.ShapeDtypeStruct((n, 16), jnp.float32)]
        scratch += [
            pltpu.VMEM((CH, 16), jnp.float32),
            pltpu.VMEM_SHARED((n, 16), jnp.float32),
            pltpu.SemaphoreType.DMA,
        ]

    @functools.partial(
        pl.kernel, mesh=mesh, out_type=out_type, scratch_types=scratch,
        name=f"sc_seg_sum_f{f}_cnt{int(with_count)}",
        compiler_params=pltpu.CompilerParams(use_tc_tiling_on_sc=False),
    )
    def k(y_hbm, src_hbm, dst_hbm, zf_hbm, zc_hbm, ones_hbm, *rest):
        if with_count:
            (p0_hbm, p1_hbm, c0_hbm, c1_hbm, src_v, dst_v, *rr) = rest
        else:
            (p0_hbm, p1_hbm, src_v, dst_v, *rr) = rest
        rows = rr[:NB]
        acc_sh = rr[NB]
        gsem = rr[NB + 1:2 * NB + 1]
        ssem = rr[2 * NB + 1:3 * NB + 1]
        if with_count:
            ones_v, cnt_sh, csem = rr[3 * NB + 1:]
        c = lax.axis_index("c")
        s = lax.axis_index("s")
        wid = c * NS + s

        row0 = pl.multiple_of(s * RPS - lax.rem(s, 8), 8)
        rsl = pl.ds(row0, RSL)

        pltpu.sync_copy(src_hbm.at[wid], src_v)
        pltpu.sync_copy(dst_hbm.at[wid], dst_v)
        pltpu.sync_copy(zf_hbm.at[rsl], acc_sh.at[rsl])
        if with_count:
            @pl.when(s == 0)
            def _():
                pltpu.sync_copy(zc_hbm, cnt_sh)
                pass
            pltpu.sync_copy(ones_hbm, ones_v)
        plsc.subcore_barrier()

        def gather_start(ch, b):
            pltpu.async_copy(y_hbm.at[src_v.at[ch]], rows[b], gsem[b])

        def gather_wait(ch, b):
            pltpu.make_async_copy(y_hbm.at[src_v.at[ch]], rows[b], gsem[b]).wait()

        def scatter_start(ch, b):
            pltpu.async_copy(rows[b], acc_sh.at[dst_v.at[ch]], ssem[b], add=True)
            if with_count:
                pltpu.async_copy(ones_v, cnt_sh.at[dst_v.at[ch]], csem, add=True)

        def scatter_wait(ch, b):
            pltpu.make_async_copy(rows[b], acc_sh.at[dst_v.at[ch]], ssem[b]).wait()
            if with_count:
                pltpu.make_async_copy(ones_v, cnt_sh.at[dst_v.at[ch]], csem).wait()

        for b in range(NB):
            gather_start(b, b)

        def body(j, _):
            j4 = NB * j
            for b in range(NB):
                gather_wait(j4 + b, b)
                scatter_start(j4 + b, b)
            for b in range(NB):
                scatter_wait(j4 + b, b)
                gather_start(j4 + NB + b, b)
            return 0

        lax.fori_loop(0, NCH // NB - 1, body, 0)
        last = NCH - NB
        for b in range(NB):
            gather_wait(last + b, b)
            scatter_start(last + b, b)
        for b in range(NB):
            scatter_wait(last + b, b)
        plsc.subcore_barrier()

        @pl.when(c == 0)
        def _():
            pltpu.sync_copy(acc_sh.at[rsl], p0_hbm.at[rsl])
            if with_count:
                @pl.when(s == 0)
                def _():
                    pltpu.sync_copy(cnt_sh, c0_hbm)
                    pass

        @pl.when(c == 1)
        def _():
            pltpu.sync_copy(acc_sh.at[rsl], p1_hbm.at[rsl])
            if with_count:
                @pl.when(s == 0)
                def _():
                    pltpu.sync_copy(cnt_sh, c1_hbm)
                    pass

    return k(y, src2d, dst2d, zf, zc, ones_c)


def _proj_body(x_ref, wl_ref, wr_ref, y_ref, r_ref):
    xb = x_ref[...]
    y_ref[...] = jnp.dot(xb, wl_ref[...], preferred_element_type=jnp.float32)
    r_ref[...] = jnp.dot(xb, wr_ref[...], preferred_element_type=jnp.float32)


def _proj(x, wl, wr):
    n, d = x.shape
    h = wl.shape[1]
    blk = 1000
    return pl.pallas_call(
        _proj_body,
        grid=(n // blk,),
        in_specs=[pl.BlockSpec((blk, d), lambda i: (i, 0)),
                  pl.BlockSpec((d, h), lambda i: (0, 0)),
                  pl.BlockSpec((d, h), lambda i: (0, 0))],
        out_specs=[pl.BlockSpec((blk, h), lambda i: (i, 0)),
                   pl.BlockSpec((blk, h), lambda i: (i, 0))],
        out_shape=[jax.ShapeDtypeStruct((n, h), jnp.float32),
                   jax.ShapeDtypeStruct((n, h), jnp.float32)],
    )(x, wl, wr)


def _norm_body(p0, p1, c0, c1, r, b, g, be, w2, x1_out, yz_out):
    nn = p0.shape[0]
    cnt = jnp.maximum(c0[...] + c1[...], 1.0)
    h = (p0[...] + p1[...]) / cnt + b[...] + r[...]
    m = jnp.sum(h, axis=0, keepdims=True) * (1.0 / nn)
    d = h - m
    v = jnp.sum(d * d, axis=0, keepdims=True) * (1.0 / nn)
    hn = d * lax.rsqrt(v + 1e-5) * g[...] + be[...]
    x1 = jnp.maximum(hn, 0.0)
    x1_out[...] = x1
    yz_out[...] = jnp.dot(x1, w2[...], preferred_element_type=jnp.float32)


def _norm(p0, p1, c0, c1, r, b, g, be, w2):
    n, f = p0.shape
    args = [p0, p1, c0, c1, r, b.reshape(1, f), g.reshape(1, f), be.reshape(1, f), w2]
    out_shape = [jax.ShapeDtypeStruct((n, f), jnp.float32),
                 jax.ShapeDtypeStruct((n, w2.shape[1]), jnp.float32)]
    return pl.pallas_call(
        _norm_body,
        in_specs=[pl.BlockSpec(a.shape, lambda: (0, 0)) for a in args],
        out_specs=[pl.BlockSpec(o.shape, lambda: (0, 0)) for o in out_shape],
        out_shape=out_shape,
    )(*args)


def _tail_body(q0, q1, c0, c1, r2, b2, g2, be2, w3, b3, w4, b4, w5, b5,
               x2_out, probs_out, logits_out, x2_s):
    i = pl.program_id(0)
    nn, blk = x2_s.shape[0], probs_out.shape[0]

    @pl.when(i == 0)
    def _():
        cnt = jnp.maximum(c0[...] + c1[...], 1.0)
        h = (q0[...] + q1[...]) / cnt + b2[...] + r2[...]
        m = jnp.sum(h, axis=0, keepdims=True) * (1.0 / nn)
        d = h - m
        v = jnp.sum(d * d, axis=0, keepdims=True) * (1.0 / nn)
        x2 = jnp.maximum(d * lax.rsqrt(v + 1e-5) * g2[...] + be2[...], 0.0)
        x2_out[...] = x2
        x2_s[...] = x2

    @pl.when(i > 0)
    def _():
        xb = x2_s[pl.ds((i - 1) * blk, blk), :]
        h = jnp.maximum(jnp.dot(xb, w3[...], preferred_element_type=jnp.float32) + b3[...], 0.0)
        h = jnp.maximum(jnp.dot(h.astype(jnp.bfloat16), w4[...], preferred_element_type=jnp.float32) + b4[...], 0.0)
        lg = jnp.dot(h, w5[...], preferred_element_type=jnp.float32) + b5[...]
        logits_out[...] = lg
        mx = jnp.max(lg, axis=1, keepdims=True)
        e = jnp.exp(lg - mx)
        probs_out[...] = e / jnp.sum(e, axis=1, keepdims=True)


def _tail(q0, q1, c0, c1, r2, b2, g2, be2, w3, b3, w4, b4, w5, b5):
    n, f = q0.shape
    mlp = w3.shape[1]
    c = w5.shape[1]
    blk = 1000
    def fixed(shape):
        return pl.BlockSpec(shape, lambda i: tuple(0 for _ in shape))
    mblk = lambda i: (jnp.maximum(i - 1, 0), 0)
    return pl.pallas_call(
        _tail_body,
        grid=(n // blk + 1,),
        in_specs=[fixed((n, f)), fixed((n, f)), fixed((n, 1)), fixed((n, 1)),
                  fixed((n, f)), fixed((1, f)), fixed((1, f)), fixed((1, f)),
                  fixed((f, mlp)), fixed((1, mlp)), fixed((mlp, mlp)),
                  fixed((1, mlp)), fixed((mlp, c)), fixed((1, c))],
        out_specs=[fixed((n, f)),
                   pl.BlockSpec((blk, c), mblk),
                   pl.BlockSpec((blk, c), mblk)],
        out_shape=[jax.ShapeDtypeStruct((n, f), jnp.float32),
                   jax.ShapeDtypeStruct((n, c), jnp.float32),
                   jax.ShapeDtypeStruct((n, c), jnp.float32)],
        scratch_shapes=[pltpu.VMEM((n, f), jnp.float32)],
    )(q0, q1, c0, c1, r2, b2.reshape(1, f), g2.reshape(1, f), be2.reshape(1, f),
      w3, b3.reshape(1, mlp), w4.astype(jnp.bfloat16), b4.reshape(1, mlp),
      w5, b5.reshape(1, c))


def kernel(x, edge_index, W1_l, b1_l, W1_r, gamma1, beta1, W2_l, b2_l, W2_r,
           gamma2, beta2, W3, b3, W4, b4, W5, b5):
    src2d = edge_index[0].reshape(NC * NS, NCH, CH)
    dst2d = edge_index[1].reshape(NC * NS, NCH, CH)
    h1 = W1_l.shape[1]
    h2 = W2_l.shape[1]
    zc = jnp.zeros((N, 16), jnp.float32)
    ones_c = jnp.ones((CH, 16), jnp.float32)

    y1, r1 = _proj(x, W1_l, W1_r)
    p0, p1, c0, c1 = _sc_agg(y1, src2d, dst2d, jnp.zeros((N, h1), jnp.float32),
                             zc, ones_c, with_count=True)
    w2cat = jnp.concatenate([W2_l, W2_r], axis=1)
    c0 = c0[:, :1]
    c1 = c1[:, :1]
    x1, yz = _norm(p0, p1, c0, c1, r1, b1_l, gamma1, beta1, w2cat)
    y2 = yz[:, :h2]
    r2 = yz[:, h2:]
    q0, q1 = _sc_agg(y2, src2d, dst2d, jnp.zeros((N, h2), jnp.float32),
                     zc, ones_c, with_count=False)
    x2, probs, logits = _tail(q0, q1, c0, c1, r2, b2_l, gamma2, beta2,
                              W3, b3, W4, b4, W5, b5)
    return probs, logits, x1, x2
